# Initial kernel scaffold; baseline (speedup 1.0000x reference)
import jax
import jax.numpy as jnp
from jax.experimental import pallas as pl


def kernel(x, edge_index, edge_attr, params):
    raise NotImplementedError("write your pallas kernel here")


# speedup vs baseline: 4.0161x; 4.0161x over previous
import functools

import jax
import jax.numpy as jnp
from jax import lax
from jax.experimental import pallas as pl
from jax.experimental.pallas import tpu as pltpu
from jax.experimental.pallas import tpu_sc as plsc

_NC = 2
_NS = 16
_NW = _NC * _NS
_K = 80
_ZR = 104


def _seg_sum_sc(table, dst_idx, src_idx, n_seg):
    n_edges = dst_idx.shape[0]
    D = table.shape[1]
    gather = src_idx is not None
    e_per_w = n_edges // _NW
    n_chunks = e_per_w // _K
    assert e_per_w % _K == 0
    rpt = (n_seg // _NS) // 8 * 8
    extra = n_seg - _NS * rpt
    assert rpt % _ZR == 0 and extra % 8 == 0 and extra <= _ZR
    zreps = rpt // _ZR

    mesh = plsc.VectorSubcoreMesh(core_axis_name="c", subcore_axis_name="s")
    scratch = [
        pltpu.VMEM((_K,), jnp.int32),
        pltpu.VMEM((_K, D), jnp.float32),
        pltpu.VMEM_SHARED((n_seg, D), jnp.float32),
        pltpu.VMEM((_ZR, D), jnp.float32),
        pltpu.SemaphoreType.DMA,
    ]
    if gather:
        scratch.insert(0, pltpu.VMEM((_K,), jnp.int32))

    def body(*refs):
        if gather:
            table_h, dst_h, src_h, out_h, sidx, didx, rows, acc, zbuf, sem = refs
        else:
            table_h, dst_h, out_h, didx, rows, acc, zbuf, sem = refs
        c = lax.axis_index("c")
        s = lax.axis_index("s")
        wid = c * _NS + s

        def zstore(t, _):
            i = t // (D // 16)
            j = t % (D // 16)
            zbuf[i, pl.ds(j * 16, 16)] = jnp.zeros((16,), jnp.float32)
            return _
        lax.fori_loop(0, _ZR * (D // 16), zstore, 0)
        for r in range(zreps):
            pltpu.sync_copy(zbuf, acc.at[pl.ds(s * rpt + r * _ZR, _ZR)])

        @pl.when(s == _NS - 1)
        def _():
            pltpu.sync_copy(zbuf.at[pl.ds(0, extra)],
                            acc.at[pl.ds(_NS * rpt, extra)])
        plsc.subcore_barrier()

        def echunk(i, _):
            off = wid * e_per_w + i * _K
            pltpu.sync_copy(dst_h.at[pl.ds(off, _K)], didx)
            if gather:
                pltpu.sync_copy(src_h.at[pl.ds(off, _K)], sidx)
                pltpu.async_copy(table_h.at[sidx], rows, sem).wait()
            else:
                pltpu.sync_copy(table_h.at[pl.ds(off, _K)], rows)
            pltpu.sync_copy(rows, acc.at[didx], add=True)
            return _
        lax.fori_loop(0, n_chunks, echunk, 0)
        plsc.subcore_barrier()

        pltpu.sync_copy(acc.at[pl.ds(s * rpt, rpt)],
                        out_h.at[c, pl.ds(s * rpt, rpt)])

        @pl.when(s == _NS - 1)
        def _():
            pltpu.sync_copy(acc.at[pl.ds(_NS * rpt, extra)],
                            out_h.at[c, pl.ds(_NS * rpt, extra)])

    operands = (table, dst_idx) + ((src_idx,) if gather else ())
    return pl.kernel(
        body,
        out_type=jax.ShapeDtypeStruct((_NC, n_seg, D), jnp.float32),
        mesh=mesh,
        scratch_types=scratch,
    )(*operands)


def _sig(v):
    return 1.0 / (1.0 + jnp.exp(-v))


_F32 = jnp.float32
_PREC = lax.Precision.HIGHEST


_BN = 2000


def _conv0_tc(Px, SeDp, W0n, W0ext):
    N = Px.shape[1]

    def body(px_ref, sedp_ref, wn_ref, wext_ref, feat_ref, sed_ref):
        sed = sedp_ref[0] + sedp_ref[1]
        sed_ref[...] = sed
        A = px_ref[0] + px_ref[1]
        agg = (jnp.dot(A, wn_ref[...], preferred_element_type=_F32, precision=_PREC)
               + jnp.dot(sed, wext_ref[...], preferred_element_type=_F32, precision=_PREC))
        feat_ref[...] = jnp.maximum(agg, 0.0)

    return pl.pallas_call(
        body,
        grid=(N // _BN,),
        in_specs=[
            pl.BlockSpec((2, _BN, 128), lambda i: (0, i, 0)),
            pl.BlockSpec((2, _BN, 128), lambda i: (0, i, 0)),
            pl.BlockSpec((128, 128), lambda i: (0, 0)),
            pl.BlockSpec((128, 128), lambda i: (0, 0)),
        ],
        out_specs=[
            pl.BlockSpec((_BN, 128), lambda i: (i, 0)),
            pl.BlockSpec((_BN, 128), lambda i: (i, 0)),
        ],
        out_shape=[jax.ShapeDtypeStruct((N, 128), _F32),
                   jax.ShapeDtypeStruct((N, 128), _F32)],
    )(Px, SeDp, W0n, W0ext)


def _conv_tc(P, feat_in, SeD, Wn, Wext):
    N = feat_in.shape[0]

    def body(p_ref, fin_ref, sed_ref, wn_ref, wext_ref, feat_ref):
        A = p_ref[0] + p_ref[1]
        agg = (jnp.dot(A, wn_ref[...], preferred_element_type=_F32, precision=_PREC)
               + jnp.dot(sed_ref[...], wext_ref[...], preferred_element_type=_F32,
                         precision=_PREC))
        feat_ref[...] = jnp.maximum(agg, 0.0) + fin_ref[...]

    return pl.pallas_call(
        body,
        grid=(N // _BN,),
        in_specs=[
            pl.BlockSpec((2, _BN, 128), lambda i: (0, i, 0)),
            pl.BlockSpec((_BN, 128), lambda i: (i, 0)),
            pl.BlockSpec((_BN, 128), lambda i: (i, 0)),
            pl.BlockSpec((128, 128), lambda i: (0, 0)),
            pl.BlockSpec((128, 128), lambda i: (0, 0)),
        ],
        out_specs=pl.BlockSpec((_BN, 128), lambda i: (i, 0)),
        out_shape=jax.ShapeDtypeStruct((N, 128), _F32),
    )(P, feat_in, SeD, Wn, Wext)


def _set2set_tc(feat, lstm, Wcls, bcls):
    (wih0, whh0, bih0, bhh0), (wih1, whh1, bih1, bhh1) = lstm

    def body(feat_in_ref,
             wih0_r, whh0_r, bih0_r, bhh0_r, wih1_r, whh1_r, bih1_r, bhh1_r,
             wcls_r, bcls_r, score_ref):
        feat = feat_in_ref[...]
        h0 = jnp.zeros((1, 128), _F32)
        h1 = jnp.zeros((1, 128), _F32)
        c0 = jnp.zeros((1, 128), _F32)
        c1 = jnp.zeros((1, 128), _F32)
        q_star = jnp.zeros((1, 256), _F32)
        for _ in range(2):
            g = (jnp.dot(q_star, wih0_r[...], preferred_element_type=_F32,
                         precision=_PREC) + bih0_r[...]
                 + jnp.dot(h0, whh0_r[...], preferred_element_type=_F32,
                           precision=_PREC) + bhh0_r[...])
            ig = _sig(g[:, 0:128])
            fg = _sig(g[:, 128:256])
            gg = jnp.tanh(g[:, 256:384])
            og = _sig(g[:, 384:512])
            c0 = fg * c0 + ig * gg
            h0 = og * jnp.tanh(c0)
            g = (jnp.dot(h0, wih1_r[...], preferred_element_type=_F32,
                         precision=_PREC) + bih1_r[...]
                 + jnp.dot(h1, whh1_r[...], preferred_element_type=_F32,
                           precision=_PREC) + bhh1_r[...])
            ig = _sig(g[:, 0:128])
            fg = _sig(g[:, 128:256])
            gg = jnp.tanh(g[:, 256:384])
            og = _sig(g[:, 384:512])
            c1 = fg * c1 + ig * gg
            h1 = og * jnp.tanh(c1)
            q = h1
            e = lax.dot_general(feat, q, (((1,), (1,)), ((), ())),
                                preferred_element_type=_F32, precision=_PREC)
            m = jnp.max(e)
            ex = jnp.exp(e - m)
            alpha = ex / jnp.sum(ex)
            r = jnp.sum(alpha * feat, axis=0, keepdims=True)
            q_star = jnp.concatenate([q, r], axis=1)
        score_ref[0, 0] = jnp.sum(q_star * wcls_r[...]) + bcls_r[0, 0]

    return pl.pallas_call(
        body,
        in_specs=[pl.BlockSpec(memory_space=pltpu.VMEM)] * 10
        + [pl.BlockSpec(memory_space=pltpu.SMEM)],
        out_specs=pl.BlockSpec(memory_space=pltpu.SMEM),
        out_shape=jax.ShapeDtypeStruct((1, 1), _F32),
    )(feat, wih0, whh0, bih0, bhh0, wih1, whh1, bih1, bhh1, Wcls, bcls)


def kernel(x, edge_index, edge_attr, params):
    N = x.shape[0]
    E = edge_index.shape[1]
    src = edge_index[0]
    dst = edge_index[1]

    ea_ext = jnp.concatenate(
        [edge_attr, jnp.ones((E, 1), _F32), jnp.zeros((E, 111), _F32)], axis=1)

    def split_w(p):
        W = p["W"]
        d_node = W.shape[0] - 16
        Wn = W[:d_node]
        Wext = jnp.concatenate(
            [W[d_node:], p["b"][None, :], jnp.zeros((111, W.shape[1]), _F32)], axis=0)
        return Wn, Wext

    SeDp = _seg_sum_sc(ea_ext, dst, None, N)

    Px = _seg_sum_sc(x, dst, src, N)
    W0n, W0ext = split_w(params["conv0"])
    feat, SeD = _conv0_tc(Px, SeDp, W0n, W0ext)

    score = jnp.zeros((1,), _F32)
    for i in range(len(params["convs"])):
        Wn, Wext = split_w(params["convs"][i])
        lstm = tuple(
            (lp["W_ih"].T, lp["W_hh"].T, lp["b_ih"][None, :], lp["b_hh"][None, :])
        for lp in params["lstms"][i])
        P = _seg_sum_sc(feat, dst, src, N)
        feat = _conv_tc(P, feat, SeD, Wn, Wext)
        s = _set2set_tc(feat, lstm,
                        params["cls"][i]["W"], params["cls"][i]["b"][None, :])
        score = score + s[0]
    return score


# speedup vs baseline: 10.4911x; 2.6122x over previous
import functools

import jax
import jax.numpy as jnp
from jax import lax
from jax.experimental import pallas as pl
from jax.experimental.pallas import tpu as pltpu
from jax.experimental.pallas import tpu_sc as plsc

_NC = 2
_NS = 16
_NW = _NC * _NS
_K = 80
_NBUF = 3


def _seg_sum_sc(table, dst_idx, src_idx, n_seg):
    n_edges = dst_idx.shape[0]
    D = table.shape[1]
    gather = src_idx is not None
    e_per_w = n_edges // _NW
    n_chunks = e_per_w // _K
    assert e_per_w % _K == 0
    rpt = (n_seg // _NS) // 8 * 8
    extra = n_seg - _NS * rpt
    assert extra % 8 == 0 and extra <= _K

    n_outer = n_chunks // _NBUF
    n_tail = n_chunks - n_outer * _NBUF

    mesh = plsc.VectorSubcoreMesh(core_axis_name="c", subcore_axis_name="s")
    scratch = (
        [pltpu.VMEM((e_per_w,), jnp.int32)]
        + ([pltpu.VMEM((e_per_w,), jnp.int32)] if gather else [])
        + [pltpu.VMEM((_K,), jnp.int32)]
        + [pltpu.VMEM((_K, D), jnp.float32) for _ in range(_NBUF)]
        + [pltpu.VMEM_SHARED((n_seg, D), jnp.float32),
           pltpu.SemaphoreType.DMA]
        + [pltpu.SemaphoreType.DMA for _ in range(_NBUF)]
    )

    def body(*refs):
        if gather:
            table_h, dst_h, src_h, out_h = refs[:4]
            didx_all, sidx_all = refs[4:6]
            rest = refs[6:]
        else:
            table_h, dst_h, out_h = refs[:3]
            didx_all = refs[3]
            sidx_all = None
            rest = refs[4:]
        didx = rest[0]
        rows = rest[1:1 + _NBUF]
        acc, psem = rest[1 + _NBUF:3 + _NBUF]
        gsem = rest[3 + _NBUF:]
        c = lax.axis_index("c")
        s = lax.axis_index("s")
        wid = c * _NS + s
        ebase = wid * e_per_w

        pltpu.async_copy(dst_h.at[pl.ds(ebase, e_per_w)], didx_all, psem)
        if gather:
            pltpu.async_copy(src_h.at[pl.ds(ebase, e_per_w)], sidx_all, psem)

        def zstore(t, carry):
            i = t // (D // 16)
            j = t % (D // 16)
            rows[0][i, pl.ds(j * 16, 16)] = jnp.zeros((16,), jnp.float32)
            return carry
        lax.fori_loop(0, _K * (D // 16), zstore, 0)
        nfull, rem = divmod(rpt, _K)
        for r in range(nfull):
            pltpu.sync_copy(rows[0], acc.at[pl.ds(s * rpt + r * _K, _K)])
        if rem:
            pltpu.sync_copy(rows[0].at[pl.ds(0, rem)],
                            acc.at[pl.ds(s * rpt + nfull * _K, rem)])

        @pl.when(s == _NS - 1)
        def _():
            pltpu.sync_copy(rows[0].at[pl.ds(0, extra)],
                            acc.at[pl.ds(_NS * rpt, extra)])

        pltpu.make_async_copy(dst_h.at[pl.ds(ebase, e_per_w)], didx_all,
                              psem).wait()
        if gather:
            pltpu.make_async_copy(src_h.at[pl.ds(ebase, e_per_w)], sidx_all,
                                  psem).wait()

        def start_gather(i, b):
            if gather:
                pltpu.async_copy(table_h.at[sidx_all.at[pl.ds(i * _K, _K)]],
                                 rows[b], gsem[b])
            else:
                pltpu.async_copy(table_h.at[pl.ds(ebase + i * _K, _K)],
                                 rows[b], gsem[b])

        for b in range(_NBUF):
            start_gather(b, b)
        plsc.subcore_barrier()

        def drain(i, b, refill):
            pltpu.make_async_copy(table_h.at[pl.ds(0, _K)], rows[b],
                                  gsem[b]).wait()
            for j in range(_K // 16):
                didx[pl.ds(j * 16, 16)] = (
                    didx_all[pl.ds(i * _K + j * 16, 16)])
            pltpu.sync_copy(rows[b], acc.at[didx], add=True)
            if refill:
                @pl.when(i + _NBUF < n_chunks)
                def _():
                    start_gather(i + _NBUF, b)

        def outer(g, carry):
            base = g * _NBUF
            for b in range(_NBUF):
                drain(base + b, b, True)
            return carry
        lax.fori_loop(0, n_outer, outer, 0)
        for t in range(n_tail):
            drain(n_outer * _NBUF + t, t, False)
        plsc.subcore_barrier()

        pltpu.sync_copy(acc.at[pl.ds(s * rpt, rpt)],
                        out_h.at[c, pl.ds(s * rpt, rpt)])

        @pl.when(s == _NS - 1)
        def _():
            pltpu.sync_copy(acc.at[pl.ds(_NS * rpt, extra)],
                            out_h.at[c, pl.ds(_NS * rpt, extra)])

    operands = (table, dst_idx) + ((src_idx,) if gather else ())
    return pl.kernel(
        body,
        out_type=jax.ShapeDtypeStruct((_NC, n_seg, D), jnp.float32),
        mesh=mesh,
        scratch_types=scratch,
    )(*operands)


def _sig(v):
    return 1.0 / (1.0 + jnp.exp(-v))


_F32 = jnp.float32
_PREC = lax.Precision.HIGHEST


_BN = 2000


def _conv0_tc(Px, SeDp, W0n, W0ext):
    N = Px.shape[1]

    def body(px_ref, sedp_ref, wn_ref, wext_ref, feat_ref, sed_ref):
        sed = sedp_ref[0] + sedp_ref[1]
        sed_ref[...] = sed
        A = px_ref[0] + px_ref[1]
        agg = (jnp.dot(A, wn_ref[...], preferred_element_type=_F32, precision=_PREC)
               + jnp.dot(sed, wext_ref[...], preferred_element_type=_F32, precision=_PREC))
        feat_ref[...] = jnp.maximum(agg, 0.0)

    return pl.pallas_call(
        body,
        grid=(N // _BN,),
        in_specs=[
            pl.BlockSpec((2, _BN, 128), lambda i: (0, i, 0)),
            pl.BlockSpec((2, _BN, 128), lambda i: (0, i, 0)),
            pl.BlockSpec((128, 128), lambda i: (0, 0)),
            pl.BlockSpec((128, 128), lambda i: (0, 0)),
        ],
        out_specs=[
            pl.BlockSpec((_BN, 128), lambda i: (i, 0)),
            pl.BlockSpec((_BN, 128), lambda i: (i, 0)),
        ],
        out_shape=[jax.ShapeDtypeStruct((N, 128), _F32),
                   jax.ShapeDtypeStruct((N, 128), _F32)],
    )(Px, SeDp, W0n, W0ext)


def _conv_tc(P, feat_in, SeD, Wn, Wext):
    N = feat_in.shape[0]

    def body(p_ref, fin_ref, sed_ref, wn_ref, wext_ref, feat_ref):
        A = p_ref[0] + p_ref[1]
        agg = (jnp.dot(A, wn_ref[...], preferred_element_type=_F32, precision=_PREC)
               + jnp.dot(sed_ref[...], wext_ref[...], preferred_element_type=_F32,
                         precision=_PREC))
        feat_ref[...] = jnp.maximum(agg, 0.0) + fin_ref[...]

    return pl.pallas_call(
        body,
        grid=(N // _BN,),
        in_specs=[
            pl.BlockSpec((2, _BN, 128), lambda i: (0, i, 0)),
            pl.BlockSpec((_BN, 128), lambda i: (i, 0)),
            pl.BlockSpec((_BN, 128), lambda i: (i, 0)),
            pl.BlockSpec((128, 128), lambda i: (0, 0)),
            pl.BlockSpec((128, 128), lambda i: (0, 0)),
        ],
        out_specs=pl.BlockSpec((_BN, 128), lambda i: (i, 0)),
        out_shape=jax.ShapeDtypeStruct((N, 128), _F32),
    )(P, feat_in, SeD, Wn, Wext)


def _set2set_tc(feat, lstm, Wcls, bcls):
    (wih0, whh0, bih0, bhh0), (wih1, whh1, bih1, bhh1) = lstm

    def body(feat_in_ref,
             wih0_r, whh0_r, bih0_r, bhh0_r, wih1_r, whh1_r, bih1_r, bhh1_r,
             wcls_r, bcls_r, score_ref):
        feat = feat_in_ref[...]
        h0 = jnp.zeros((1, 128), _F32)
        h1 = jnp.zeros((1, 128), _F32)
        c0 = jnp.zeros((1, 128), _F32)
        c1 = jnp.zeros((1, 128), _F32)
        q_star = jnp.zeros((1, 256), _F32)
        for _ in range(2):
            g = (jnp.dot(q_star, wih0_r[...], preferred_element_type=_F32,
                         precision=_PREC) + bih0_r[...]
                 + jnp.dot(h0, whh0_r[...], preferred_element_type=_F32,
                           precision=_PREC) + bhh0_r[...])
            ig = _sig(g[:, 0:128])
            fg = _sig(g[:, 128:256])
            gg = jnp.tanh(g[:, 256:384])
            og = _sig(g[:, 384:512])
            c0 = fg * c0 + ig * gg
            h0 = og * jnp.tanh(c0)
            g = (jnp.dot(h0, wih1_r[...], preferred_element_type=_F32,
                         precision=_PREC) + bih1_r[...]
                 + jnp.dot(h1, whh1_r[...], preferred_element_type=_F32,
                           precision=_PREC) + bhh1_r[...])
            ig = _sig(g[:, 0:128])
            fg = _sig(g[:, 128:256])
            gg = jnp.tanh(g[:, 256:384])
            og = _sig(g[:, 384:512])
            c1 = fg * c1 + ig * gg
            h1 = og * jnp.tanh(c1)
            q = h1
            e = lax.dot_general(feat, q, (((1,), (1,)), ((), ())),
                                preferred_element_type=_F32, precision=_PREC)
            m = jnp.max(e)
            ex = jnp.exp(e - m)
            alpha = ex / jnp.sum(ex)
            r = jnp.sum(alpha * feat, axis=0, keepdims=True)
            q_star = jnp.concatenate([q, r], axis=1)
        score_ref[0, 0] = jnp.sum(q_star * wcls_r[...]) + bcls_r[0, 0]

    return pl.pallas_call(
        body,
        in_specs=[pl.BlockSpec(memory_space=pltpu.VMEM)] * 10
        + [pl.BlockSpec(memory_space=pltpu.SMEM)],
        out_specs=pl.BlockSpec(memory_space=pltpu.SMEM),
        out_shape=jax.ShapeDtypeStruct((1, 1), _F32),
    )(feat, wih0, whh0, bih0, bhh0, wih1, whh1, bih1, bhh1, Wcls, bcls)


def kernel(x, edge_index, edge_attr, params):
    N = x.shape[0]
    E = edge_index.shape[1]
    src = edge_index[0]
    dst = edge_index[1]

    ea_ext = jnp.concatenate(
        [edge_attr, jnp.ones((E, 1), _F32), jnp.zeros((E, 111), _F32)], axis=1)

    def split_w(p):
        W = p["W"]
        d_node = W.shape[0] - 16
        Wn = W[:d_node]
        Wext = jnp.concatenate(
            [W[d_node:], p["b"][None, :], jnp.zeros((111, W.shape[1]), _F32)], axis=0)
        return Wn, Wext

    SeDp = _seg_sum_sc(ea_ext, dst, None, N)

    Px = _seg_sum_sc(x, dst, src, N)
    W0n, W0ext = split_w(params["conv0"])
    feat, SeD = _conv0_tc(Px, SeDp, W0n, W0ext)

    score = jnp.zeros((1,), _F32)
    for i in range(len(params["convs"])):
        Wn, Wext = split_w(params["convs"][i])
        lstm = tuple(
            (lp["W_ih"].T, lp["W_hh"].T, lp["b_ih"][None, :], lp["b_hh"][None, :])
        for lp in params["lstms"][i])
        P = _seg_sum_sc(feat, dst, src, N)
        feat = _conv_tc(P, feat, SeD, Wn, Wext)
        s = _set2set_tc(feat, lstm,
                        params["cls"][i]["W"], params["cls"][i]["b"][None, :])
        score = score + s[0]
    return score


# speedup vs baseline: 10.6303x; 1.0133x over previous
import functools

import jax
import jax.numpy as jnp
from jax import lax
from jax.experimental import pallas as pl
from jax.experimental.pallas import tpu as pltpu
from jax.experimental.pallas import tpu_sc as plsc

_NC = 2
_NS = 16
_NW = _NC * _NS
_K = 80
_NBUF = 4
_NI = 8


def _seg_sum_sc(table, dst_idx, src_idx, n_seg):
    n_edges = dst_idx.shape[0]
    D = table.shape[1]
    gather = src_idx is not None
    e_per_w = n_edges // _NW
    n_chunks = e_per_w // _K
    assert e_per_w % _K == 0
    rpt = (n_seg // _NS) // 8 * 8
    extra = n_seg - _NS * rpt
    assert extra % 8 == 0 and extra <= _K

    n_outer = n_chunks // _NI
    n_tail = n_chunks - n_outer * _NI

    mesh = plsc.VectorSubcoreMesh(core_axis_name="c", subcore_axis_name="s")
    scratch = (
        [pltpu.VMEM((_K,), jnp.int32) for _ in range(_NI)]
        + ([pltpu.VMEM((_K,), jnp.int32) for _ in range(_NI)]
           if gather else [])
        + [pltpu.VMEM((_K, D), jnp.float32) for _ in range(_NBUF)]
        + [pltpu.VMEM_SHARED((n_seg, D), jnp.float32)]
        + [pltpu.SemaphoreType.DMA for _ in range(_NI)]
        + [pltpu.SemaphoreType.DMA for _ in range(_NBUF)]
    )

    def body(*refs):
        if gather:
            table_h, dst_h, src_h, out_h = refs[:4]
            rest = refs[4:]
            didx_b = rest[:_NI]
            sidx_b = rest[_NI:2 * _NI]
            rest = rest[2 * _NI:]
        else:
            table_h, dst_h, out_h = refs[:3]
            rest = refs[3:]
            didx_b = rest[:_NI]
            sidx_b = None
            rest = rest[_NI:]
        rows = rest[:_NBUF]
        acc = rest[_NBUF]
        isem = rest[_NBUF + 1:_NBUF + 1 + _NI]
        gsem = rest[_NBUF + 1 + _NI:]
        c = lax.axis_index("c")
        s = lax.axis_index("s")
        wid = c * _NS + s
        ebase = wid * e_per_w

        def start_idx(i, q):
            pltpu.async_copy(dst_h.at[pl.ds(ebase + i * _K, _K)], didx_b[q],
                             isem[q])
            if gather:
                pltpu.async_copy(src_h.at[pl.ds(ebase + i * _K, _K)],
                                 sidx_b[q], isem[q])

        def wait_idx(q):
            pltpu.make_async_copy(dst_h.at[pl.ds(0, _K)], didx_b[q],
                                  isem[q]).wait()
            if gather:
                pltpu.make_async_copy(src_h.at[pl.ds(0, _K)], sidx_b[q],
                                      isem[q]).wait()

        def start_gather(i, b, q):
            if gather:
                pltpu.async_copy(table_h.at[sidx_b[q]], rows[b], gsem[b])
            else:
                pltpu.async_copy(table_h.at[pl.ds(ebase + i * _K, _K)],
                                 rows[b], gsem[b])

        for q in range(_NI):
            start_idx(q, q)

        def zstore(t, carry):
            i = t // (D // 16)
            j = t % (D // 16)
            rows[0][i, pl.ds(j * 16, 16)] = jnp.zeros((16,), jnp.float32)
            return carry
        lax.fori_loop(0, _K * (D // 16), zstore, 0)
        nfull, rem = divmod(rpt, _K)
        for r in range(nfull):
            pltpu.sync_copy(rows[0], acc.at[pl.ds(s * rpt + r * _K, _K)])
        if rem:
            pltpu.sync_copy(rows[0].at[pl.ds(0, rem)],
                            acc.at[pl.ds(s * rpt + nfull * _K, rem)])

        @pl.when(s == _NS - 1)
        def _():
            pltpu.sync_copy(rows[0].at[pl.ds(0, extra)],
                            acc.at[pl.ds(_NS * rpt, extra)])

        for b in range(_NBUF):
            wait_idx(b)
            start_gather(b, b, b)
        plsc.subcore_barrier()

        def drain(i, b, q):
            pltpu.make_async_copy(table_h.at[pl.ds(0, _K)], rows[b],
                                  gsem[b]).wait()
            pltpu.sync_copy(rows[b], acc.at[didx_b[q]], add=True)

            @pl.when(i + _NI < n_chunks)
            def _():
                start_idx(i + _NI, q)

            @pl.when(i + _NBUF < n_chunks)
            def _():
                wait_idx((q + _NBUF) % _NI)
                start_gather(i + _NBUF, b, (q + _NBUF) % _NI)

        def outer(g, carry):
            base = g * _NI
            for k in range(_NI):
                drain(base + k, k % _NBUF, k)
            return carry
        lax.fori_loop(0, n_outer, outer, 0)
        for t in range(n_tail):
            i = n_outer * _NI + t
            drain(i, i % _NBUF, i % _NI)
        plsc.subcore_barrier()

        pltpu.sync_copy(acc.at[pl.ds(s * rpt, rpt)],
                        out_h.at[c, pl.ds(s * rpt, rpt)])

        @pl.when(s == _NS - 1)
        def _():
            pltpu.sync_copy(acc.at[pl.ds(_NS * rpt, extra)],
                            out_h.at[c, pl.ds(_NS * rpt, extra)])

    operands = (table, dst_idx) + ((src_idx,) if gather else ())
    return pl.kernel(
        body,
        out_type=jax.ShapeDtypeStruct((_NC, n_seg, D), jnp.float32),
        mesh=mesh,
        scratch_types=scratch,
    )(*operands)


def _sig(v):
    return 1.0 / (1.0 + jnp.exp(-v))


_F32 = jnp.float32
_PREC = lax.Precision.HIGHEST


_BN = 2000


def _conv0_tc(Px, SeDp, W0n, W0ext):
    N = Px.shape[1]

    def body(px_ref, sedp_ref, wn_ref, wext_ref, feat_ref, sed_ref):
        sed = sedp_ref[0] + sedp_ref[1]
        sed_ref[...] = sed
        A = px_ref[0] + px_ref[1]
        agg = (jnp.dot(A, wn_ref[...], preferred_element_type=_F32, precision=_PREC)
               + jnp.dot(sed, wext_ref[...], preferred_element_type=_F32, precision=_PREC))
        feat_ref[...] = jnp.maximum(agg, 0.0)

    return pl.pallas_call(
        body,
        grid=(N // _BN,),
        in_specs=[
            pl.BlockSpec((2, _BN, 128), lambda i: (0, i, 0)),
            pl.BlockSpec((2, _BN, 128), lambda i: (0, i, 0)),
            pl.BlockSpec((128, 128), lambda i: (0, 0)),
            pl.BlockSpec((128, 128), lambda i: (0, 0)),
        ],
        out_specs=[
            pl.BlockSpec((_BN, 128), lambda i: (i, 0)),
            pl.BlockSpec((_BN, 128), lambda i: (i, 0)),
        ],
        out_shape=[jax.ShapeDtypeStruct((N, 128), _F32),
                   jax.ShapeDtypeStruct((N, 128), _F32)],
    )(Px, SeDp, W0n, W0ext)


def _conv_tc(P, feat_in, SeD, Wn, Wext):
    N = feat_in.shape[0]

    def body(p_ref, fin_ref, sed_ref, wn_ref, wext_ref, feat_ref):
        A = p_ref[0] + p_ref[1]
        agg = (jnp.dot(A, wn_ref[...], preferred_element_type=_F32, precision=_PREC)
               + jnp.dot(sed_ref[...], wext_ref[...], preferred_element_type=_F32,
                         precision=_PREC))
        feat_ref[...] = jnp.maximum(agg, 0.0) + fin_ref[...]

    return pl.pallas_call(
        body,
        grid=(N // _BN,),
        in_specs=[
            pl.BlockSpec((2, _BN, 128), lambda i: (0, i, 0)),
            pl.BlockSpec((_BN, 128), lambda i: (i, 0)),
            pl.BlockSpec((_BN, 128), lambda i: (i, 0)),
            pl.BlockSpec((128, 128), lambda i: (0, 0)),
            pl.BlockSpec((128, 128), lambda i: (0, 0)),
        ],
        out_specs=pl.BlockSpec((_BN, 128), lambda i: (i, 0)),
        out_shape=jax.ShapeDtypeStruct((N, 128), _F32),
    )(P, feat_in, SeD, Wn, Wext)


def _set2set_tc(feat, lstm, Wcls, bcls):
    (wih0, whh0, bih0, bhh0), (wih1, whh1, bih1, bhh1) = lstm

    def body(feat_in_ref,
             wih0_r, whh0_r, bih0_r, bhh0_r, wih1_r, whh1_r, bih1_r, bhh1_r,
             wcls_r, bcls_r, score_ref):
        feat = feat_in_ref[...]
        h0 = jnp.zeros((1, 128), _F32)
        h1 = jnp.zeros((1, 128), _F32)
        c0 = jnp.zeros((1, 128), _F32)
        c1 = jnp.zeros((1, 128), _F32)
        q_star = jnp.zeros((1, 256), _F32)
        for _ in range(2):
            g = (jnp.dot(q_star, wih0_r[...], preferred_element_type=_F32,
                         precision=_PREC) + bih0_r[...]
                 + jnp.dot(h0, whh0_r[...], preferred_element_type=_F32,
                           precision=_PREC) + bhh0_r[...])
            ig = _sig(g[:, 0:128])
            fg = _sig(g[:, 128:256])
            gg = jnp.tanh(g[:, 256:384])
            og = _sig(g[:, 384:512])
            c0 = fg * c0 + ig * gg
            h0 = og * jnp.tanh(c0)
            g = (jnp.dot(h0, wih1_r[...], preferred_element_type=_F32,
                         precision=_PREC) + bih1_r[...]
                 + jnp.dot(h1, whh1_r[...], preferred_element_type=_F32,
                           precision=_PREC) + bhh1_r[...])
            ig = _sig(g[:, 0:128])
            fg = _sig(g[:, 128:256])
            gg = jnp.tanh(g[:, 256:384])
            og = _sig(g[:, 384:512])
            c1 = fg * c1 + ig * gg
            h1 = og * jnp.tanh(c1)
            q = h1
            e = lax.dot_general(feat, q, (((1,), (1,)), ((), ())),
                                preferred_element_type=_F32, precision=_PREC)
            m = jnp.max(e)
            ex = jnp.exp(e - m)
            alpha = ex / jnp.sum(ex)
            r = jnp.sum(alpha * feat, axis=0, keepdims=True)
            q_star = jnp.concatenate([q, r], axis=1)
        score_ref[0, 0] = jnp.sum(q_star * wcls_r[...]) + bcls_r[0, 0]

    return pl.pallas_call(
        body,
        in_specs=[pl.BlockSpec(memory_space=pltpu.VMEM)] * 10
        + [pl.BlockSpec(memory_space=pltpu.SMEM)],
        out_specs=pl.BlockSpec(memory_space=pltpu.SMEM),
        out_shape=jax.ShapeDtypeStruct((1, 1), _F32),
    )(feat, wih0, whh0, bih0, bhh0, wih1, whh1, bih1, bhh1, Wcls, bcls)


def kernel(x, edge_index, edge_attr, params):
    N = x.shape[0]
    E = edge_index.shape[1]
    src = edge_index[0]
    dst = edge_index[1]

    ea_ext = jnp.concatenate(
        [edge_attr, jnp.ones((E, 1), _F32), jnp.zeros((E, 111), _F32)], axis=1)

    def split_w(p):
        W = p["W"]
        d_node = W.shape[0] - 16
        Wn = W[:d_node]
        Wext = jnp.concatenate(
            [W[d_node:], p["b"][None, :], jnp.zeros((111, W.shape[1]), _F32)], axis=0)
        return Wn, Wext

    SeDp = _seg_sum_sc(ea_ext, dst, None, N)

    Px = _seg_sum_sc(x, dst, src, N)
    W0n, W0ext = split_w(params["conv0"])
    feat, SeD = _conv0_tc(Px, SeDp, W0n, W0ext)

    score = jnp.zeros((1,), _F32)
    for i in range(len(params["convs"])):
        Wn, Wext = split_w(params["convs"][i])
        lstm = tuple(
            (lp["W_ih"].T, lp["W_hh"].T, lp["b_ih"][None, :], lp["b_hh"][None, :])
        for lp in params["lstms"][i])
        P = _seg_sum_sc(feat, dst, src, N)
        feat = _conv_tc(P, feat, SeD, Wn, Wext)
        s = _set2set_tc(feat, lstm,
                        params["cls"][i]["W"], params["cls"][i]["b"][None, :])
        score = score + s[0]
    return score


# speedup vs baseline: 10.8578x; 1.0214x over previous
import functools

import jax
import jax.numpy as jnp
from jax import lax
from jax.experimental import pallas as pl
from jax.experimental.pallas import tpu as pltpu
from jax.experimental.pallas import tpu_sc as plsc

_NC = 2
_NS = 16
_NW = _NC * _NS
_K = 80
_NBUF = 3


def _seg_sum_sc(table, dst_idx, src_idx, n_seg):
    n_edges = dst_idx.shape[0]
    D = table.shape[1]
    gather = src_idx is not None
    e_per_w = n_edges // _NW
    n_chunks = e_per_w // _K
    assert e_per_w % _K == 0
    rpt = (n_seg // _NS) // 8 * 8
    extra = n_seg - _NS * rpt
    assert extra % 8 == 0 and extra <= _K

    n_outer = n_chunks // _NBUF
    n_tail = n_chunks - n_outer * _NBUF

    mesh = plsc.VectorSubcoreMesh(core_axis_name="c", subcore_axis_name="s")
    scratch = (
        [pltpu.VMEM((e_per_w,), jnp.int32)]
        + ([pltpu.VMEM((e_per_w,), jnp.int32)] if gather else [])
        + [pltpu.VMEM((_K,), jnp.int32)]
        + [pltpu.VMEM((_K, D), jnp.float32) for _ in range(_NBUF)]
        + [pltpu.VMEM_SHARED((n_seg, D), jnp.float32),
           pltpu.SemaphoreType.DMA]
        + [pltpu.SemaphoreType.DMA for _ in range(_NBUF)]
    )

    def body(*refs):
        if gather:
            table_h, dst_h, src_h, out_h = refs[:4]
            didx_all, sidx_all = refs[4:6]
            rest = refs[6:]
        else:
            table_h, dst_h, out_h = refs[:3]
            didx_all = refs[3]
            sidx_all = None
            rest = refs[4:]
        didx = rest[0]
        rows = rest[1:1 + _NBUF]
        acc, psem = rest[1 + _NBUF:3 + _NBUF]
        gsem = rest[3 + _NBUF:]
        c = lax.axis_index("c")
        s = lax.axis_index("s")
        wid = c * _NS + s
        ebase = wid * e_per_w

        pltpu.async_copy(dst_h.at[pl.ds(ebase, e_per_w)], didx_all, psem)
        if gather:
            pltpu.async_copy(src_h.at[pl.ds(ebase, e_per_w)], sidx_all, psem)

        def zstore(t, carry):
            i = t // (D // 16)
            j = t % (D // 16)
            rows[0][i, pl.ds(j * 16, 16)] = jnp.zeros((16,), jnp.float32)
            return carry
        lax.fori_loop(0, _K * (D // 16), zstore, 0)
        nfull, rem = divmod(rpt, _K)
        for r in range(nfull):
            pltpu.sync_copy(rows[0], acc.at[pl.ds(s * rpt + r * _K, _K)])
        if rem:
            pltpu.sync_copy(rows[0].at[pl.ds(0, rem)],
                            acc.at[pl.ds(s * rpt + nfull * _K, rem)])

        @pl.when(s == _NS - 1)
        def _():
            pltpu.sync_copy(rows[0].at[pl.ds(0, extra)],
                            acc.at[pl.ds(_NS * rpt, extra)])

        pltpu.make_async_copy(dst_h.at[pl.ds(ebase, e_per_w)], didx_all,
                              psem).wait()
        if gather:
            pltpu.make_async_copy(src_h.at[pl.ds(ebase, e_per_w)], sidx_all,
                                  psem).wait()

        def start_gather(i, b):
            if gather:
                pltpu.async_copy(table_h.at[sidx_all.at[pl.ds(i * _K, _K)]],
                                 rows[b], gsem[b])
            else:
                pltpu.async_copy(table_h.at[pl.ds(ebase + i * _K, _K)],
                                 rows[b], gsem[b])

        for b in range(_NBUF):
            start_gather(b, b)
        plsc.subcore_barrier()

        def drain(i, b, refill):
            pltpu.make_async_copy(table_h.at[pl.ds(0, _K)], rows[b],
                                  gsem[b]).wait()
            for j in range(_K // 16):
                didx[pl.ds(j * 16, 16)] = (
                    didx_all[pl.ds(i * _K + j * 16, 16)])
            pltpu.sync_copy(rows[b], acc.at[didx], add=True)
            if refill:
                @pl.when(i + _NBUF < n_chunks)
                def _():
                    start_gather(i + _NBUF, b)

        def outer(g, carry):
            base = g * _NBUF
            for b in range(_NBUF):
                drain(base + b, b, True)
            return carry
        lax.fori_loop(0, n_outer, outer, 0)
        for t in range(n_tail):
            drain(n_outer * _NBUF + t, t, False)
        plsc.subcore_barrier()

        pltpu.sync_copy(acc.at[pl.ds(s * rpt, rpt)],
                        out_h.at[c, pl.ds(s * rpt, rpt)])

        @pl.when(s == _NS - 1)
        def _():
            pltpu.sync_copy(acc.at[pl.ds(_NS * rpt, extra)],
                            out_h.at[c, pl.ds(_NS * rpt, extra)])

    operands = (table, dst_idx) + ((src_idx,) if gather else ())
    return pl.kernel(
        body,
        out_type=jax.ShapeDtypeStruct((_NC, n_seg, D), jnp.float32),
        mesh=mesh,
        scratch_types=scratch,
    )(*operands)


def _sig(v):
    return 1.0 / (1.0 + jnp.exp(-v))


_F32 = jnp.float32
_PREC = lax.Precision.DEFAULT


_BN = 2000


def _conv0_tc(Px, SeDp, W0n, W0ext):
    N = Px.shape[1]

    def body(px_ref, sedp_ref, wn_ref, wext_ref, feat_ref, sed_ref):
        sed = sedp_ref[0] + sedp_ref[1]
        sed_ref[...] = sed
        A = px_ref[0] + px_ref[1]
        agg = (jnp.dot(A, wn_ref[...], preferred_element_type=_F32, precision=_PREC)
               + jnp.dot(sed, wext_ref[...], preferred_element_type=_F32, precision=_PREC))
        feat_ref[...] = jnp.maximum(agg, 0.0)

    return pl.pallas_call(
        body,
        grid=(N // _BN,),
        in_specs=[
            pl.BlockSpec((2, _BN, 128), lambda i: (0, i, 0)),
            pl.BlockSpec((2, _BN, 128), lambda i: (0, i, 0)),
            pl.BlockSpec((128, 128), lambda i: (0, 0)),
            pl.BlockSpec((128, 128), lambda i: (0, 0)),
        ],
        out_specs=[
            pl.BlockSpec((_BN, 128), lambda i: (i, 0)),
            pl.BlockSpec((_BN, 128), lambda i: (i, 0)),
        ],
        out_shape=[jax.ShapeDtypeStruct((N, 128), _F32),
                   jax.ShapeDtypeStruct((N, 128), _F32)],
    )(Px, SeDp, W0n, W0ext)


def _conv_tc(P, feat_in, SeD, Wn, Wext):
    N = feat_in.shape[0]

    def body(p_ref, fin_ref, sed_ref, wn_ref, wext_ref, feat_ref):
        A = p_ref[0] + p_ref[1]
        agg = (jnp.dot(A, wn_ref[...], preferred_element_type=_F32, precision=_PREC)
               + jnp.dot(sed_ref[...], wext_ref[...], preferred_element_type=_F32,
                         precision=_PREC))
        feat_ref[...] = jnp.maximum(agg, 0.0) + fin_ref[...]

    return pl.pallas_call(
        body,
        grid=(N // _BN,),
        in_specs=[
            pl.BlockSpec((2, _BN, 128), lambda i: (0, i, 0)),
            pl.BlockSpec((_BN, 128), lambda i: (i, 0)),
            pl.BlockSpec((_BN, 128), lambda i: (i, 0)),
            pl.BlockSpec((128, 128), lambda i: (0, 0)),
            pl.BlockSpec((128, 128), lambda i: (0, 0)),
        ],
        out_specs=pl.BlockSpec((_BN, 128), lambda i: (i, 0)),
        out_shape=jax.ShapeDtypeStruct((N, 128), _F32),
    )(P, feat_in, SeD, Wn, Wext)


def _set2set_tc(feat, lstm, Wcls, bcls):
    (wih0, whh0, bih0, bhh0), (wih1, whh1, bih1, bhh1) = lstm

    def body(feat_in_ref,
             wih0_r, whh0_r, bih0_r, bhh0_r, wih1_r, whh1_r, bih1_r, bhh1_r,
             wcls_r, bcls_r, score_ref):
        feat = feat_in_ref[...]
        h0 = jnp.zeros((1, 128), _F32)
        h1 = jnp.zeros((1, 128), _F32)
        c0 = jnp.zeros((1, 128), _F32)
        c1 = jnp.zeros((1, 128), _F32)
        q_star = jnp.zeros((1, 256), _F32)
        for _ in range(2):
            g = (jnp.dot(q_star, wih0_r[...], preferred_element_type=_F32,
                         precision=_PREC) + bih0_r[...]
                 + jnp.dot(h0, whh0_r[...], preferred_element_type=_F32,
                           precision=_PREC) + bhh0_r[...])
            ig = _sig(g[:, 0:128])
            fg = _sig(g[:, 128:256])
            gg = jnp.tanh(g[:, 256:384])
            og = _sig(g[:, 384:512])
            c0 = fg * c0 + ig * gg
            h0 = og * jnp.tanh(c0)
            g = (jnp.dot(h0, wih1_r[...], preferred_element_type=_F32,
                         precision=_PREC) + bih1_r[...]
                 + jnp.dot(h1, whh1_r[...], preferred_element_type=_F32,
                           precision=_PREC) + bhh1_r[...])
            ig = _sig(g[:, 0:128])
            fg = _sig(g[:, 128:256])
            gg = jnp.tanh(g[:, 256:384])
            og = _sig(g[:, 384:512])
            c1 = fg * c1 + ig * gg
            h1 = og * jnp.tanh(c1)
            q = h1
            e = lax.dot_general(feat, q, (((1,), (1,)), ((), ())),
                                preferred_element_type=_F32, precision=_PREC)
            m = jnp.max(e)
            ex = jnp.exp(e - m)
            alpha = ex / jnp.sum(ex)
            r = jnp.sum(alpha * feat, axis=0, keepdims=True)
            q_star = jnp.concatenate([q, r], axis=1)
        score_ref[0, 0] = jnp.sum(q_star * wcls_r[...]) + bcls_r[0, 0]

    return pl.pallas_call(
        body,
        in_specs=[pl.BlockSpec(memory_space=pltpu.VMEM)] * 10
        + [pl.BlockSpec(memory_space=pltpu.SMEM)],
        out_specs=pl.BlockSpec(memory_space=pltpu.SMEM),
        out_shape=jax.ShapeDtypeStruct((1, 1), _F32),
    )(feat, wih0, whh0, bih0, bhh0, wih1, whh1, bih1, bhh1, Wcls, bcls)


def kernel(x, edge_index, edge_attr, params):
    N = x.shape[0]
    E = edge_index.shape[1]
    src = edge_index[0]
    dst = edge_index[1]

    ea_ext = jnp.concatenate(
        [edge_attr, jnp.ones((E, 1), _F32), jnp.zeros((E, 111), _F32)], axis=1)

    def split_w(p):
        W = p["W"]
        d_node = W.shape[0] - 16
        Wn = W[:d_node]
        Wext = jnp.concatenate(
            [W[d_node:], p["b"][None, :], jnp.zeros((111, W.shape[1]), _F32)], axis=0)
        return Wn, Wext

    SeDp = _seg_sum_sc(ea_ext, dst, None, N)

    Px = _seg_sum_sc(x, dst, src, N)
    W0n, W0ext = split_w(params["conv0"])
    feat, SeD = _conv0_tc(Px, SeDp, W0n, W0ext)

    score = jnp.zeros((1,), _F32)
    for i in range(len(params["convs"])):
        Wn, Wext = split_w(params["convs"][i])
        lstm = tuple(
            (lp["W_ih"].T, lp["W_hh"].T, lp["b_ih"][None, :], lp["b_hh"][None, :])
        for lp in params["lstms"][i])
        P = _seg_sum_sc(feat, dst, src, N)
        feat = _conv_tc(P, feat, SeD, Wn, Wext)
        s = _set2set_tc(feat, lstm,
                        params["cls"][i]["W"], params["cls"][i]["b"][None, :])
        score = score + s[0]
    return score
